# Initial kernel scaffold; baseline (speedup 1.0000x reference)
import jax
import jax.numpy as jnp
from jax.experimental import pallas as pl


def kernel(h, edge_index, lin_w, att_src, att_dst, bias):
    raise NotImplementedError("write your pallas kernel here")


# speedup vs baseline: 20.1205x; 20.1205x over previous
import functools

import jax
import jax.numpy as jnp
from jax import lax
from jax.experimental import pallas as pl
from jax.experimental.pallas import tpu as pltpu
from jax.experimental.pallas import tpu_sc as plsc


def _proj_body(h_ref, w_ref, asv_ref, adv_ref, hp_ref, as_ref, ad_ref):
    hp = lax.dot_general(h_ref[...], w_ref[...], (((1,), (1,)), ((), ())),
                         preferred_element_type=jnp.float32)
    hp_ref[...] = hp
    as_ref[...] = jnp.sum(hp * asv_ref[...], axis=1, keepdims=True)
    ad_ref[...] = jnp.sum(hp * adv_ref[...], axis=1, keepdims=True)


def _project(h, lin_w, asv, adv, blk=400):
    n, in_c = h.shape
    out_c = lin_w.shape[0]
    grid = n // blk
    return pl.pallas_call(
        _proj_body,
        grid=(grid,),
        in_specs=[
            pl.BlockSpec((blk, in_c), lambda i: (i, 0)),
            pl.BlockSpec((out_c, in_c), lambda i: (0, 0)),
            pl.BlockSpec((1, out_c), lambda i: (0, 0)),
            pl.BlockSpec((1, out_c), lambda i: (0, 0)),
        ],
        out_specs=[
            pl.BlockSpec((blk, out_c), lambda i: (i, 0)),
            pl.BlockSpec((blk, 1), lambda i: (i, 0)),
            pl.BlockSpec((blk, 1), lambda i: (i, 0)),
        ],
        out_shape=[
            jax.ShapeDtypeStruct((n, out_c), jnp.float32),
            jax.ShapeDtypeStruct((n, 1), jnp.float32),
            jax.ShapeDtypeStruct((n, 1), jnp.float32),
        ],
    )(h, lin_w, asv, adv)


_K = 128
_NW = 32
_NPAD = 10240


def _make_edge_kernel(n, out_c, e_real, et_pad):
    nchunk = et_pad // _K
    mesh = plsc.VectorSubcoreMesh(core_axis_name="c", subcore_axis_name="s")
    rows_per_tile = n // 16
    dpad_per_tile = _NPAD // 16

    @functools.partial(
        pl.kernel,
        mesh=mesh,
        compiler_params=pltpu.CompilerParams(needs_layout_passes=False),
        out_type=[
            jax.ShapeDtypeStruct((2, _NPAD, out_c), jnp.float32),
            jax.ShapeDtypeStruct((2, _NPAD), jnp.float32),
        ],
        scratch_types=[
            pltpu.VMEM((n,), jnp.float32),
            pltpu.VMEM((n,), jnp.float32),
            pltpu.VMEM((_K,), jnp.int32),
            pltpu.VMEM((_K,), jnp.int32),
            pltpu.VMEM((_K,), jnp.float32),
            pltpu.VMEM((_K, out_c), jnp.float32),
            pltpu.VMEM((16, out_c), jnp.float32),
            pltpu.VMEM((dpad_per_tile,), jnp.float32),
            pltpu.VMEM_SHARED((_NPAD, out_c), jnp.float32),
            pltpu.VMEM_SHARED((_NPAD,), jnp.float32),
            pltpu.SemaphoreType.DMA,
        ],
    )
    def edge_kernel(src_hbm, dst_hbm, asv_hbm, adv_hbm, hp_hbm,
                    outp_hbm, den_hbm,
                    as_v, ad_v, src_v, dst_v, w_v, rows_v, zb_v, zd_v,
                    acc_sh, den_sh, sem):
        c = lax.axis_index("c")
        s = lax.axis_index("s")
        tid = c * 16 + s

        zeros16 = jnp.zeros((16,), jnp.float32)
        for i in range(16):
            for cc in range(out_c // 16):
                zb_v[i, pl.ds(cc * 16, 16)] = zeros16
        for i in range(dpad_per_tile // 16):
            zd_v[pl.ds(i * 16, 16)] = zeros16
        for b in range(dpad_per_tile // 16):
            pltpu.sync_copy(zb_v, acc_sh.at[pl.ds(s * dpad_per_tile + b * 16, 16), :])
        pltpu.sync_copy(zd_v, den_sh.at[pl.ds(s * dpad_per_tile, dpad_per_tile)])

        pltpu.sync_copy(asv_hbm, as_v)
        pltpu.sync_copy(adv_hbm, ad_v)
        plsc.subcore_barrier()

        base = tid * et_pad

        def chunk(t, carry):
            off = base + t * _K
            pltpu.sync_copy(src_hbm.at[pl.ds(off, _K)], src_v)
            pltpu.sync_copy(dst_hbm.at[pl.ds(off, _K)], dst_v)
            cp = pltpu.async_copy(hp_hbm.at[src_v], rows_v, sem)
            for g in range(_K // 16):
                s16 = src_v[pl.ds(g * 16, 16)]
                d16 = dst_v[pl.ds(g * 16, 16)]
                x = plsc.load_gather(as_v, [s16]) + plsc.load_gather(ad_v, [d16])
                w16 = jnp.exp(1.0 / (1.0 + jnp.exp(-x)))
                gid = off + g * 16 + lax.iota(jnp.int32, 16)
                w16 = jnp.where(gid < e_real, w16, 0.0)
                w_v[pl.ds(g * 16, 16)] = w16
            cp.wait()

            def grp(g, _):
                w16 = w_v[pl.ds(g * 16, 16)]
                for j in range(16):
                    wj = w16[j]
                    r = g * 16 + j
                    for cc in range(out_c // 16):
                        rows_v[r, pl.ds(cc * 16, 16)] = rows_v[r, pl.ds(cc * 16, 16)] * wj
                return 0

            lax.fori_loop(0, _K // 16, grp, 0)
            pltpu.sync_copy(rows_v, acc_sh.at[dst_v], add=True)
            pltpu.sync_copy(w_v, den_sh.at[dst_v], add=True)
            return carry

        lax.fori_loop(0, nchunk, chunk, 0)
        plsc.subcore_barrier()

        pltpu.sync_copy(acc_sh.at[pl.ds(s * dpad_per_tile, dpad_per_tile), :],
                        outp_hbm.at[c, pl.ds(s * dpad_per_tile, dpad_per_tile), :])
        pltpu.sync_copy(den_sh.at[pl.ds(s * dpad_per_tile, dpad_per_tile)],
                        den_hbm.at[c, pl.ds(s * dpad_per_tile, dpad_per_tile)])

    return edge_kernel


def _norm_body(p0_ref, p1_ref, d0_ref, d1_ref, hp_ref, as_ref, ad_ref, b_ref, out_ref):
    x = as_ref[...] + ad_ref[...]
    ws = jnp.exp(1.0 / (1.0 + jnp.exp(-x)))
    dt = d0_ref[...] + d1_ref[...] + ws
    num = p0_ref[...] + p1_ref[...] + ws * hp_ref[...]
    out_ref[...] = num / dt + b_ref[...]


def _normalize(p0, p1, d0, d1, hp, as2, ad2, bias2, blk=400):
    n, out_c = hp.shape
    grid = n // blk
    col = pl.BlockSpec((blk, 1), lambda i: (i, 0))
    mat = pl.BlockSpec((blk, out_c), lambda i: (i, 0))
    return pl.pallas_call(
        _norm_body,
        grid=(grid,),
        in_specs=[mat, mat, col, col, mat, col, col,
                  pl.BlockSpec((1, out_c), lambda i: (0, 0))],
        out_specs=mat,
        out_shape=jax.ShapeDtypeStruct((n, out_c), jnp.float32),
    )(p0, p1, d0, d1, hp, as2, ad2, bias2)


def kernel(h, edge_index, lin_w, att_src, att_dst, bias):
    n, in_c = h.shape
    out_c = lin_w.shape[0]
    e = edge_index.shape[1]

    asv = att_src.reshape(1, out_c)
    adv = att_dst.reshape(1, out_c)
    hp, as2, ad2 = _project(h, lin_w, asv, adv)

    et_pad = -(-e // (_NW * _K)) * _K
    e_pad = et_pad * _NW
    src = edge_index[0]
    dst = edge_index[1]
    pad = jnp.zeros((e_pad - e,), dtype=jnp.int32)
    src_p = jnp.concatenate([src, pad])
    dst_p = jnp.concatenate([dst, pad])

    edge_kernel = _make_edge_kernel(n, out_c, e, et_pad)
    outp, den = edge_kernel(src_p, dst_p, as2.reshape(n), ad2.reshape(n), hp)

    out = _normalize(outp[0, :n], outp[1, :n],
                     den[0, :n].reshape(n, 1), den[1, :n].reshape(n, 1),
                     hp, as2, ad2, bias.reshape(1, out_c))
    return out
